# baseline (speedup 1.0000x reference)
import functools

import jax
import jax.numpy as jnp
from jax import lax
from jax.experimental import pallas as pl
from jax.experimental.pallas import tpu as pltpu
from jax.experimental.pallas import tpu_sc as plsc

N = 100000
E = 3200000
SHEET_COND = 0.01
SHEET_EXP = 1.25
CHAN_COND = 0.1
CHAN_EXP = 3.0
BED_STEP = 0.1
CAV_SPACING = 2.0
CLOSURE = 5e-25
PMC = 7.5e-08
CP = 4220.0
RHO_W = 1000.0
RHO_I = 917.0
G = 9.81
SEC_PER_A = 31556926.0
LATENT = 334000.0
RWG = RHO_W * G
RIG = RHO_I * G

NC = 2
NS = 16
NW = NC * NS

CH = 128
N_CHUNKS = E // CH
BASE_UNITS = N_CHUNKS // NW
EXTRA = N_CHUNKS - BASE_UNITS * NW

NSL = 6240
NTAIL = N - NS * NSL
NTB = NS * NSL

WSL = 3120
WTAIL = N - NW * WSL
WTB = NW * WSL


def _rsqrt(x):
    i = lax.bitcast_convert_type(x, jnp.int32)
    i = jnp.int32(0x5F3759DF) - (i >> 1)
    y = lax.bitcast_convert_type(i, jnp.float32)
    for _ in range(3):
        y = y * (1.5 - 0.5 * x * y * y)
    return y


def _edge_math(pth, ptt, hh, ht, wph, wpt, neh, net, s_ch, u_sl):
    grad = pth - ptt
    absg = jnp.abs(grad) + 1e-8
    hl = 0.5 * (hh + ht)
    hs = jnp.maximum(hl, 1e-30)
    hp = hl * _rsqrt(_rsqrt(hs))
    sheet_q = (-SHEET_COND) * hp * _rsqrt(absg) * grad
    chan_q = (-CHAN_COND) * (s_ch * s_ch * s_ch) * grad
    diss = jnp.abs(chan_q * grad) + jnp.abs(CAV_SPACING * sheet_q * grad)
    pgrad = wph - wpt
    cond = (s_ch > 0) | ((pgrad * sheet_q) > 0)
    totq = jnp.where(cond, chan_q + CAV_SPACING * sheet_q, chan_q)
    sens = (-PMC * CP * RHO_W) * totq * pgrad
    nl = 0.5 * (neh + net)
    nlc = jnp.maximum(nl, 0.0)
    ccl = CLOSURE * s_ch * (nlc * nlc * nlc)
    melt = (diss - sens) * (1.0 / (RHO_I * LATENT))
    dsdt = melt - ccl
    aslide = jnp.abs(u_sl) * (1.0 / SEC_PER_A)
    return dsdt, aslide


def _edge_body(pot_hbm, h_hbm, bed_hbm, ice_hbm, chan_hbm, slid_hbm, tail_hbm, head_hbm,
               dq_out, sl0_out, sl1_out, dg0_out, dg1_out,
               pot_sh, hsh_sh, wp_sh, ne_sh, slide_sh, deg_sh,
               potb, bedb, iceb, wpb, neb,
               ih, it, sv, uv, gph, gpt, ghh, ght, gwh, gwt, gnh, gnt, dq, sl, ones,
               seml, semg):
    c = lax.axis_index("c")
    s = lax.axis_index("s")
    w = s * NC + c

    def _stage(nb, nsl, iters):
        pltpu.sync_copy(pot_hbm.at[pl.ds(nb, nsl)], potb.at[pl.ds(0, nsl)])
        pltpu.sync_copy(bed_hbm.at[pl.ds(nb, nsl)], bedb.at[pl.ds(0, nsl)])
        pltpu.sync_copy(ice_hbm.at[pl.ds(nb, nsl)], iceb.at[pl.ds(0, nsl)])

        def nbody(i, carry):
            dsl = pl.ds(pl.multiple_of(i * 16, 16), 16)
            p = potb[dsl]
            bp = RWG * bedb[dsl]
            wpb[dsl] = p - bp
            neb[dsl] = bp + RIG * iceb[dsl] - p
            return carry

        lax.fori_loop(0, iters, nbody, 0)
        pltpu.sync_copy(potb.at[pl.ds(0, nsl)], pot_sh.at[pl.ds(nb, nsl)])
        pltpu.sync_copy(wpb.at[pl.ds(0, nsl)], wp_sh.at[pl.ds(nb, nsl)])
        pltpu.sync_copy(neb.at[pl.ds(0, nsl)], ne_sh.at[pl.ds(nb, nsl)])
        pltpu.sync_copy(h_hbm.at[pl.ds(nb, nsl)], bedb.at[pl.ds(0, nsl)])
        pltpu.sync_copy(bedb.at[pl.ds(0, nsl)], hsh_sh.at[pl.ds(nb, nsl)])

        def zbody(i, carry):
            dsl = pl.ds(pl.multiple_of(i * 16, 16), 16)
            wpb[dsl] = jnp.zeros((16,), jnp.float32)
            return carry

        lax.fori_loop(0, iters, zbody, 0)
        pltpu.sync_copy(wpb.at[pl.ds(0, nsl)], slide_sh.at[pl.ds(nb, nsl)])
        pltpu.sync_copy(wpb.at[pl.ds(0, nsl)], deg_sh.at[pl.ds(nb, nsl)])

    _stage(pl.multiple_of(s * NSL, 32), NSL, NSL // 16)

    @pl.when(s == 0)
    def _():
        _stage(NTB, NTAIL, NTAIL // 16)

    for i in range(CH // 16):
        ones[pl.ds(i * 16, 16)] = jnp.ones((16,), jnp.float32)

    plsc.subcore_barrier()

    n_units = jnp.where(w < EXTRA, BASE_UNITS + 1, BASE_UNITS)

    def ebody(j, carry):
        base = pl.multiple_of((w + NW * j) * CH, CH)
        cp = [pltpu.async_copy(tail_hbm.at[pl.ds(base, CH)], it, seml),
              pltpu.async_copy(head_hbm.at[pl.ds(base, CH)], ih, seml),
              pltpu.async_copy(chan_hbm.at[pl.ds(base, CH)], sv, seml),
              pltpu.async_copy(slid_hbm.at[pl.ds(base, CH)], uv, seml)]
        for x in cp:
            x.wait()
        gs = [pltpu.async_copy(pot_sh.at[ih], gph, semg),
              pltpu.async_copy(pot_sh.at[it], gpt, semg),
              pltpu.async_copy(hsh_sh.at[ih], ghh, semg),
              pltpu.async_copy(hsh_sh.at[it], ght, semg),
              pltpu.async_copy(wp_sh.at[ih], gwh, semg),
              pltpu.async_copy(wp_sh.at[it], gwt, semg),
              pltpu.async_copy(ne_sh.at[ih], gnh, semg),
              pltpu.async_copy(ne_sh.at[it], gnt, semg)]
        for x in gs:
            x.wait()
        for i in range(CH // 16):
            dsl = pl.ds(i * 16, 16)
            dsdt, aslide = _edge_math(gph[dsl], gpt[dsl], ghh[dsl], ght[dsl],
                                      gwh[dsl], gwt[dsl], gnh[dsl], gnt[dsl],
                                      sv[dsl], uv[dsl])
            dq[dsl] = dsdt
            sl[dsl] = aslide
        pltpu.sync_copy(dq, dq_out.at[pl.ds(base, CH)])
        pltpu.sync_copy(sl, slide_sh.at[ih], add=True)
        pltpu.sync_copy(sl, slide_sh.at[it], add=True)
        pltpu.sync_copy(ones, deg_sh.at[ih], add=True)
        pltpu.sync_copy(ones, deg_sh.at[it], add=True)
        return carry

    lax.fori_loop(0, n_units, ebody, 0)

    plsc.subcore_barrier()

    def _wb(nb, nsl, slide_out, deg_out):
        pltpu.sync_copy(slide_sh.at[pl.ds(nb, nsl)], potb.at[pl.ds(0, nsl)])
        pltpu.sync_copy(potb.at[pl.ds(0, nsl)], slide_out.at[pl.ds(nb, nsl)])
        pltpu.sync_copy(deg_sh.at[pl.ds(nb, nsl)], wpb.at[pl.ds(0, nsl)])
        pltpu.sync_copy(wpb.at[pl.ds(0, nsl)], deg_out.at[pl.ds(nb, nsl)])

    nb_main = pl.multiple_of(s * NSL, 32)

    @pl.when(c == 0)
    def _():
        _wb(nb_main, NSL, sl0_out, dg0_out)

    @pl.when(c == 1)
    def _():
        _wb(nb_main, NSL, sl1_out, dg1_out)

    @pl.when((s == 0) & (c == 0))
    def _():
        _wb(NTB, NTAIL, sl0_out, dg0_out)

    @pl.when((s == 0) & (c == 1))
    def _():
        _wb(NTB, NTAIL, sl1_out, dg1_out)


def _node_body(pot_hbm, h_hbm, bed_hbm, ice_hbm, sl0_hbm, sl1_hbm, dg0_hbm, dg1_hbm,
               dh_out,
               potb, hb, bedb, iceb, sp0, sp1, dp0, dp1, dhb):
    c = lax.axis_index("c")
    s = lax.axis_index("s")
    w = s * NC + c

    def _run(nb, nsl, iters):
        pltpu.sync_copy(pot_hbm.at[pl.ds(nb, nsl)], potb.at[pl.ds(0, nsl)])
        pltpu.sync_copy(h_hbm.at[pl.ds(nb, nsl)], hb.at[pl.ds(0, nsl)])
        pltpu.sync_copy(bed_hbm.at[pl.ds(nb, nsl)], bedb.at[pl.ds(0, nsl)])
        pltpu.sync_copy(ice_hbm.at[pl.ds(nb, nsl)], iceb.at[pl.ds(0, nsl)])
        pltpu.sync_copy(sl0_hbm.at[pl.ds(nb, nsl)], sp0.at[pl.ds(0, nsl)])
        pltpu.sync_copy(sl1_hbm.at[pl.ds(nb, nsl)], sp1.at[pl.ds(0, nsl)])
        pltpu.sync_copy(dg0_hbm.at[pl.ds(nb, nsl)], dp0.at[pl.ds(0, nsl)])
        pltpu.sync_copy(dg1_hbm.at[pl.ds(nb, nsl)], dp1.at[pl.ds(0, nsl)])

        def nbody(i, carry):
            dsl = pl.ds(pl.multiple_of(i * 16, 16), 16)
            p = potb[dsl]
            h = hb[dsl]
            ne = RWG * bedb[dsl] + RIG * iceb[dsl] - p
            nec = jnp.maximum(ne, 0.0)
            scl = CLOSURE * h * (nec * nec * nec)
            dg = dp0[dsl] + dp1[dsl]
            sn = (sp0[dsl] + sp1[dsl]) / jnp.maximum(dg, 1.0)
            opening = jnp.where(h < BED_STEP,
                                sn * (BED_STEP - h) * (1.0 / CAV_SPACING), 0.0)
            dhb[dsl] = opening - scl
            return carry

        lax.fori_loop(0, iters, nbody, 0)
        pltpu.sync_copy(dhb.at[pl.ds(0, nsl)], dh_out.at[pl.ds(nb, nsl)])

    _run(pl.multiple_of(w * WSL, 16), WSL, WSL // 16)

    @pl.when(w == 0)
    def _():
        _run(WTB, WTAIL, WTAIL // 16)


_MESH = plsc.VectorSubcoreMesh(core_axis_name="c", subcore_axis_name="s")

_edge_kernel = functools.partial(
    pl.kernel,
    out_type=(jax.ShapeDtypeStruct((E,), jnp.float32),
              jax.ShapeDtypeStruct((N,), jnp.float32),
              jax.ShapeDtypeStruct((N,), jnp.float32),
              jax.ShapeDtypeStruct((N,), jnp.float32),
              jax.ShapeDtypeStruct((N,), jnp.float32)),
    mesh=_MESH,
    scratch_types=(
        pltpu.VMEM_SHARED((N,), jnp.float32),
        pltpu.VMEM_SHARED((N,), jnp.float32),
        pltpu.VMEM_SHARED((N,), jnp.float32),
        pltpu.VMEM_SHARED((N,), jnp.float32),
        pltpu.VMEM_SHARED((N,), jnp.float32),
        pltpu.VMEM_SHARED((N,), jnp.float32),
        pltpu.VMEM((NSL,), jnp.float32),
        pltpu.VMEM((NSL,), jnp.float32),
        pltpu.VMEM((NSL,), jnp.float32),
        pltpu.VMEM((NSL,), jnp.float32),
        pltpu.VMEM((NSL,), jnp.float32),
        pltpu.VMEM((CH,), jnp.int32),
        pltpu.VMEM((CH,), jnp.int32),
        pltpu.VMEM((CH,), jnp.float32),
        pltpu.VMEM((CH,), jnp.float32),
        pltpu.VMEM((CH,), jnp.float32),
        pltpu.VMEM((CH,), jnp.float32),
        pltpu.VMEM((CH,), jnp.float32),
        pltpu.VMEM((CH,), jnp.float32),
        pltpu.VMEM((CH,), jnp.float32),
        pltpu.VMEM((CH,), jnp.float32),
        pltpu.VMEM((CH,), jnp.float32),
        pltpu.VMEM((CH,), jnp.float32),
        pltpu.VMEM((CH,), jnp.float32),
        pltpu.VMEM((CH,), jnp.float32),
        pltpu.VMEM((CH,), jnp.float32),
        pltpu.SemaphoreType.DMA,
        pltpu.SemaphoreType.DMA,
    ),
)(_edge_body)

_node_kernel = functools.partial(
    pl.kernel,
    out_type=jax.ShapeDtypeStruct((N,), jnp.float32),
    mesh=_MESH,
    scratch_types=tuple([pltpu.VMEM((WSL,), jnp.float32)] * 9),
)(_node_body)


def kernel(potential, sheet_thickness, channel_size, sliding_velocity,
           bedrock_elevation, ice_thickness, edge_index):
    tail = edge_index[0]
    head = edge_index[1]
    dsdt, sl0, sl1, dg0, dg1 = _edge_kernel(
        potential, sheet_thickness, bedrock_elevation, ice_thickness,
        channel_size, sliding_velocity, tail, head)
    dhdt = _node_kernel(potential, sheet_thickness, bedrock_elevation,
                        ice_thickness, sl0, sl1, dg0, dg1)
    return jnp.concatenate([dhdt, dsdt])


# speedup vs baseline: 1.2767x; 1.2767x over previous
import functools

import jax
import jax.numpy as jnp
from jax import lax
from jax.experimental import pallas as pl
from jax.experimental.pallas import tpu as pltpu
from jax.experimental.pallas import tpu_sc as plsc

N = 100000
E = 3200000
SHEET_COND = 0.01
SHEET_EXP = 1.25
CHAN_COND = 0.1
CHAN_EXP = 3.0
BED_STEP = 0.1
CAV_SPACING = 2.0
CLOSURE = 5e-25
PMC = 7.5e-08
CP = 4220.0
RHO_W = 1000.0
RHO_I = 917.0
G = 9.81
SEC_PER_A = 31556926.0
LATENT = 334000.0
RWG = RHO_W * G
RIG = RHO_I * G

NC = 2
NS = 16
NW = NC * NS

CH = 128
RB = 4
UE = RB * CH
N_UNITS = E // UE
BASE_UNITS = N_UNITS // NW
EXTRA = N_UNITS - BASE_UNITS * NW

NSL = 6240
NTAIL = N - NS * NSL
NTB = NS * NSL

WSL = 3120
WTAIL = N - NW * WSL
WTB = NW * WSL


def _rsqrt(x):
    i = lax.bitcast_convert_type(x, jnp.int32)
    i = jnp.int32(0x5F3759DF) - (i >> 1)
    y = lax.bitcast_convert_type(i, jnp.float32)
    for _ in range(3):
        y = y * (1.5 - 0.5 * x * y * y)
    return y


def _edge_math(pth, ptt, hh, ht, wph, wpt, neh, net, s_ch, u_sl):
    grad = pth - ptt
    absg = jnp.abs(grad) + 1e-8
    hl = 0.5 * (hh + ht)
    hs = jnp.maximum(hl, 1e-30)
    hp = hl * _rsqrt(_rsqrt(hs))
    sheet_q = (-SHEET_COND) * hp * _rsqrt(absg) * grad
    chan_q = (-CHAN_COND) * (s_ch * s_ch * s_ch) * grad
    diss = jnp.abs(chan_q * grad) + jnp.abs(CAV_SPACING * sheet_q * grad)
    pgrad = wph - wpt
    cond = (s_ch > 0) | ((pgrad * sheet_q) > 0)
    totq = jnp.where(cond, chan_q + CAV_SPACING * sheet_q, chan_q)
    sens = (-PMC * CP * RHO_W) * totq * pgrad
    nl = 0.5 * (neh + net)
    nlc = jnp.maximum(nl, 0.0)
    ccl = CLOSURE * s_ch * (nlc * nlc * nlc)
    melt = (diss - sens) * (1.0 / (RHO_I * LATENT))
    dsdt = melt - ccl
    aslide = jnp.abs(u_sl) * (1.0 / SEC_PER_A)
    return dsdt, aslide


def _edge_body(pot_hbm, h_hbm, bed_hbm, ice_hbm, chan_hbm, slid_hbm, tail_hbm, head_hbm,
               dq_out, sl0_out, sl1_out, dg0_out, dg1_out,
               pot_sh, hsh_sh, wp_sh, ne_sh, slide_sh, deg_sh,
               b1, b2, b3,
               ih0, ih1, ih2, ih3, it0, it1, it2, it3,
               sv, uv, gph, gpt, ghh, ght, gwh, gwt, gnh, gnt, dq, sl, ones,
               seml, semg, sems):
    c = lax.axis_index("c")
    s = lax.axis_index("s")
    w = s * NC + c
    ihs = (ih0, ih1, ih2, ih3)
    its = (it0, it1, it2, it3)

    def _stage(nb, nsl, iters):
        pltpu.sync_copy(pot_hbm.at[pl.ds(nb, nsl)], b3.at[pl.ds(0, nsl)])
        pltpu.sync_copy(bed_hbm.at[pl.ds(nb, nsl)], b1.at[pl.ds(0, nsl)])
        pltpu.sync_copy(ice_hbm.at[pl.ds(nb, nsl)], b2.at[pl.ds(0, nsl)])

        def nbody(i, carry):
            dsl = pl.ds(pl.multiple_of(i * 16, 16), 16)
            p = b3[dsl]
            bp = RWG * b1[dsl]
            icv = b2[dsl]
            b1[dsl] = p - bp
            b2[dsl] = bp + RIG * icv - p
            return carry

        lax.fori_loop(0, iters, nbody, 0)
        pltpu.sync_copy(b3.at[pl.ds(0, nsl)], pot_sh.at[pl.ds(nb, nsl)])
        pltpu.sync_copy(b1.at[pl.ds(0, nsl)], wp_sh.at[pl.ds(nb, nsl)])
        pltpu.sync_copy(b2.at[pl.ds(0, nsl)], ne_sh.at[pl.ds(nb, nsl)])
        pltpu.sync_copy(h_hbm.at[pl.ds(nb, nsl)], b1.at[pl.ds(0, nsl)])
        pltpu.sync_copy(b1.at[pl.ds(0, nsl)], hsh_sh.at[pl.ds(nb, nsl)])

        def zbody(i, carry):
            dsl = pl.ds(pl.multiple_of(i * 16, 16), 16)
            b1[dsl] = jnp.zeros((16,), jnp.float32)
            return carry

        lax.fori_loop(0, iters, zbody, 0)
        pltpu.sync_copy(b1.at[pl.ds(0, nsl)], slide_sh.at[pl.ds(nb, nsl)])
        pltpu.sync_copy(b1.at[pl.ds(0, nsl)], deg_sh.at[pl.ds(nb, nsl)])

    _stage(pl.multiple_of(s * NSL, 32), NSL, NSL // 16)

    @pl.when(s == 0)
    def _():
        _stage(NTB, NTAIL, NTAIL // 16)

    for i in range(CH // 16):
        ones[pl.ds(i * 16, 16)] = jnp.ones((16,), jnp.float32)

    plsc.subcore_barrier()

    n_iters = jnp.where(w < EXTRA, BASE_UNITS + 1, BASE_UNITS)
    start = BASE_UNITS * w + jnp.minimum(w, EXTRA)

    def ebody(j, carry):
        base = pl.multiple_of((start + j) * UE, UE)
        cp = [pltpu.async_copy(chan_hbm.at[pl.ds(base, UE)], sv, seml),
              pltpu.async_copy(slid_hbm.at[pl.ds(base, UE)], uv, seml)]
        for r in range(RB):
            brow = pl.ds(base + r * CH, CH)
            cp.append(pltpu.async_copy(head_hbm.at[brow], ihs[r], seml))
            cp.append(pltpu.async_copy(tail_hbm.at[brow], its[r], seml))
        for x in cp:
            x.wait()
        gs = []
        for r in range(RB):
            drow = pl.ds(r * CH, CH)
            gs += [pltpu.async_copy(pot_sh.at[ihs[r]], gph.at[drow], semg),
                   pltpu.async_copy(pot_sh.at[its[r]], gpt.at[drow], semg),
                   pltpu.async_copy(hsh_sh.at[ihs[r]], ghh.at[drow], semg),
                   pltpu.async_copy(hsh_sh.at[its[r]], ght.at[drow], semg),
                   pltpu.async_copy(wp_sh.at[ihs[r]], gwh.at[drow], semg),
                   pltpu.async_copy(wp_sh.at[its[r]], gwt.at[drow], semg),
                   pltpu.async_copy(ne_sh.at[ihs[r]], gnh.at[drow], semg),
                   pltpu.async_copy(ne_sh.at[its[r]], gnt.at[drow], semg)]
        for x in gs:
            x.wait()
        for r in range(RB):
            for i in range(CH // 16):
                dsl = pl.ds(r * CH + i * 16, 16)
                dsl_r = pl.ds(i * 16, 16)
                dsdt, aslide = _edge_math(gph[dsl], gpt[dsl],
                                          ghh[dsl], ght[dsl],
                                          gwh[dsl], gwt[dsl], gnh[dsl], gnt[dsl],
                                          sv[dsl], uv[dsl])
                dq[dsl] = dsdt
                sl[r, dsl_r] = aslide
        ocp = pltpu.async_copy(dq, dq_out.at[pl.ds(base, UE)], seml)
        for r in range(RB):
            pltpu.sync_copy(sl.at[r], slide_sh.at[ihs[r]], add=True)
            pltpu.sync_copy(sl.at[r], slide_sh.at[its[r]], add=True)
            pltpu.sync_copy(ones, deg_sh.at[ihs[r]], add=True)
            pltpu.sync_copy(ones, deg_sh.at[its[r]], add=True)
        ocp.wait()
        return carry

    lax.fori_loop(0, n_iters, ebody, 0)

    plsc.subcore_barrier()

    def _wb(nb, nsl, slide_out, deg_out):
        pltpu.sync_copy(slide_sh.at[pl.ds(nb, nsl)], b1.at[pl.ds(0, nsl)])
        pltpu.sync_copy(b1.at[pl.ds(0, nsl)], slide_out.at[pl.ds(nb, nsl)])
        pltpu.sync_copy(deg_sh.at[pl.ds(nb, nsl)], b2.at[pl.ds(0, nsl)])
        pltpu.sync_copy(b2.at[pl.ds(0, nsl)], deg_out.at[pl.ds(nb, nsl)])

    nb_main = pl.multiple_of(s * NSL, 32)

    @pl.when(c == 0)
    def _():
        _wb(nb_main, NSL, sl0_out, dg0_out)

    @pl.when(c == 1)
    def _():
        _wb(nb_main, NSL, sl1_out, dg1_out)

    @pl.when((s == 0) & (c == 0))
    def _():
        _wb(NTB, NTAIL, sl0_out, dg0_out)

    @pl.when((s == 0) & (c == 1))
    def _():
        _wb(NTB, NTAIL, sl1_out, dg1_out)


def _node_body(pot_hbm, h_hbm, bed_hbm, ice_hbm, sl0_hbm, sl1_hbm, dg0_hbm, dg1_hbm,
               dh_out,
               potb, hb, bedb, iceb, sp0, sp1, dp0, dp1, dhb):
    c = lax.axis_index("c")
    s = lax.axis_index("s")
    w = s * NC + c

    def _run(nb, nsl, iters):
        pltpu.sync_copy(pot_hbm.at[pl.ds(nb, nsl)], potb.at[pl.ds(0, nsl)])
        pltpu.sync_copy(h_hbm.at[pl.ds(nb, nsl)], hb.at[pl.ds(0, nsl)])
        pltpu.sync_copy(bed_hbm.at[pl.ds(nb, nsl)], bedb.at[pl.ds(0, nsl)])
        pltpu.sync_copy(ice_hbm.at[pl.ds(nb, nsl)], iceb.at[pl.ds(0, nsl)])
        pltpu.sync_copy(sl0_hbm.at[pl.ds(nb, nsl)], sp0.at[pl.ds(0, nsl)])
        pltpu.sync_copy(sl1_hbm.at[pl.ds(nb, nsl)], sp1.at[pl.ds(0, nsl)])
        pltpu.sync_copy(dg0_hbm.at[pl.ds(nb, nsl)], dp0.at[pl.ds(0, nsl)])
        pltpu.sync_copy(dg1_hbm.at[pl.ds(nb, nsl)], dp1.at[pl.ds(0, nsl)])

        def nbody(i, carry):
            dsl = pl.ds(pl.multiple_of(i * 16, 16), 16)
            p = potb[dsl]
            h = hb[dsl]
            ne = RWG * bedb[dsl] + RIG * iceb[dsl] - p
            nec = jnp.maximum(ne, 0.0)
            scl = CLOSURE * h * (nec * nec * nec)
            dg = dp0[dsl] + dp1[dsl]
            sn = (sp0[dsl] + sp1[dsl]) / jnp.maximum(dg, 1.0)
            opening = jnp.where(h < BED_STEP,
                                sn * (BED_STEP - h) * (1.0 / CAV_SPACING), 0.0)
            dhb[dsl] = opening - scl
            return carry

        lax.fori_loop(0, iters, nbody, 0)
        pltpu.sync_copy(dhb.at[pl.ds(0, nsl)], dh_out.at[pl.ds(nb, nsl)])

    _run(pl.multiple_of(w * WSL, 16), WSL, WSL // 16)

    @pl.when(w == 0)
    def _():
        _run(WTB, WTAIL, WTAIL // 16)


_MESH = plsc.VectorSubcoreMesh(core_axis_name="c", subcore_axis_name="s")

_edge_kernel = functools.partial(
    pl.kernel,
    out_type=(jax.ShapeDtypeStruct((E,), jnp.float32),
              jax.ShapeDtypeStruct((N,), jnp.float32),
              jax.ShapeDtypeStruct((N,), jnp.float32),
              jax.ShapeDtypeStruct((N,), jnp.float32),
              jax.ShapeDtypeStruct((N,), jnp.float32)),
    mesh=_MESH,
    scratch_types=(
        pltpu.VMEM_SHARED((N,), jnp.float32),
        pltpu.VMEM_SHARED((N,), jnp.float32),
        pltpu.VMEM_SHARED((N,), jnp.float32),
        pltpu.VMEM_SHARED((N,), jnp.float32),
        pltpu.VMEM_SHARED((N,), jnp.float32),
        pltpu.VMEM_SHARED((N,), jnp.float32),
        pltpu.VMEM((NSL,), jnp.float32),
        pltpu.VMEM((NSL,), jnp.float32),
        pltpu.VMEM((NSL,), jnp.float32),
        pltpu.VMEM((CH,), jnp.int32),
        pltpu.VMEM((CH,), jnp.int32),
        pltpu.VMEM((CH,), jnp.int32),
        pltpu.VMEM((CH,), jnp.int32),
        pltpu.VMEM((CH,), jnp.int32),
        pltpu.VMEM((CH,), jnp.int32),
        pltpu.VMEM((CH,), jnp.int32),
        pltpu.VMEM((CH,), jnp.int32),
        pltpu.VMEM((UE,), jnp.float32),
        pltpu.VMEM((UE,), jnp.float32),
        pltpu.VMEM((UE,), jnp.float32),
        pltpu.VMEM((UE,), jnp.float32),
        pltpu.VMEM((UE,), jnp.float32),
        pltpu.VMEM((UE,), jnp.float32),
        pltpu.VMEM((UE,), jnp.float32),
        pltpu.VMEM((UE,), jnp.float32),
        pltpu.VMEM((UE,), jnp.float32),
        pltpu.VMEM((UE,), jnp.float32),
        pltpu.VMEM((UE,), jnp.float32),
        pltpu.VMEM((RB, CH), jnp.float32),
        pltpu.VMEM((CH,), jnp.float32),
        pltpu.SemaphoreType.DMA,
        pltpu.SemaphoreType.DMA,
        pltpu.SemaphoreType.DMA,
    ),
)(_edge_body)

_node_kernel = functools.partial(
    pl.kernel,
    out_type=jax.ShapeDtypeStruct((N,), jnp.float32),
    mesh=_MESH,
    scratch_types=tuple([pltpu.VMEM((WSL,), jnp.float32)] * 9),
)(_node_body)


def kernel(potential, sheet_thickness, channel_size, sliding_velocity,
           bedrock_elevation, ice_thickness, edge_index):
    tail = edge_index[0]
    head = edge_index[1]
    dsdt, sl0, sl1, dg0, dg1 = _edge_kernel(
        potential, sheet_thickness, bedrock_elevation, ice_thickness,
        channel_size, sliding_velocity, tail, head)
    dhdt = _node_kernel(potential, sheet_thickness, bedrock_elevation,
                        ice_thickness, sl0, sl1, dg0, dg1)
    return jnp.concatenate([dhdt, dsdt])


# speedup vs baseline: 1.5257x; 1.1950x over previous
import functools

import jax
import jax.numpy as jnp
from jax import lax
from jax.experimental import pallas as pl
from jax.experimental.pallas import tpu as pltpu
from jax.experimental.pallas import tpu_sc as plsc

N = 100000
E = 3200000
SHEET_COND = 0.01
SHEET_EXP = 1.25
CHAN_COND = 0.1
CHAN_EXP = 3.0
BED_STEP = 0.1
CAV_SPACING = 2.0
CLOSURE = 5e-25
PMC = 7.5e-08
CP = 4220.0
RHO_W = 1000.0
RHO_I = 917.0
G = 9.81
SEC_PER_A = 31556926.0
LATENT = 334000.0
RWG = RHO_W * G
RIG = RHO_I * G

NC = 2
NS = 16
NW = NC * NS

CH = 128
RB = 4
UE = RB * CH
N_UNITS = E // UE
BASE_UNITS = N_UNITS // NW
EXTRA = N_UNITS - BASE_UNITS * NW

NSL = 6240
NTAIL = N - NS * NSL
NTB = NS * NSL

WSL = 3120
WTAIL = N - NW * WSL
WTB = NW * WSL


def _rsqrt(x):
    i = lax.bitcast_convert_type(x, jnp.int32)
    i = jnp.int32(0x5F3759DF) - (i >> 1)
    y = lax.bitcast_convert_type(i, jnp.float32)
    for _ in range(2):
        y = y * (1.5 - 0.5 * x * y * y)
    return y


def _edge_math(pth, ptt, hh, ht, wph, wpt, neh, net, s_ch, u_sl):
    grad = pth - ptt
    absg = jnp.abs(grad) + 1e-8
    hl = 0.5 * (hh + ht)
    hs = jnp.maximum(hl, 1e-30)
    hp = hl * _rsqrt(_rsqrt(hs))
    sheet_q = (-SHEET_COND) * hp * _rsqrt(absg) * grad
    chan_q = (-CHAN_COND) * (s_ch * s_ch * s_ch) * grad
    diss = jnp.abs(chan_q * grad) + jnp.abs(CAV_SPACING * sheet_q * grad)
    pgrad = wph - wpt
    cond = (s_ch > 0) | ((pgrad * sheet_q) > 0)
    totq = jnp.where(cond, chan_q + CAV_SPACING * sheet_q, chan_q)
    sens = (-PMC * CP * RHO_W) * totq * pgrad
    nl = 0.5 * (neh + net)
    nlc = jnp.maximum(nl, 0.0)
    ccl = CLOSURE * s_ch * (nlc * nlc * nlc)
    melt = (diss - sens) * (1.0 / (RHO_I * LATENT))
    dsdt = melt - ccl
    aslide = jnp.abs(u_sl) * (1.0 / SEC_PER_A)
    return dsdt, aslide


def _edge_body(pot_hbm, h_hbm, bed_hbm, ice_hbm, chan_hbm, slid_hbm, tail_hbm, head_hbm,
               dq_out, sl0_out, sl1_out, dg0_out, dg1_out,
               pot_sh, hsh_sh, wp_sh, ne_sh, slide_sh, deg_sh,
               b1, b2, b3,
               ih0, ih1, ih2, ih3, it0, it1, it2, it3,
               sv, uv, gph, gpt, ghh, ght, gwh, gwt, gnh, gnt, dq, sl, ones,
               seml, semg, sems):
    c = lax.axis_index("c")
    s = lax.axis_index("s")
    w = s * NC + c
    ihs = (ih0, ih1, ih2, ih3)
    its = (it0, it1, it2, it3)

    def _stage(nb, nsl, iters):
        pltpu.sync_copy(pot_hbm.at[pl.ds(nb, nsl)], b3.at[pl.ds(0, nsl)])
        pltpu.sync_copy(bed_hbm.at[pl.ds(nb, nsl)], b1.at[pl.ds(0, nsl)])
        pltpu.sync_copy(ice_hbm.at[pl.ds(nb, nsl)], b2.at[pl.ds(0, nsl)])

        def nbody(i, carry):
            dsl = pl.ds(pl.multiple_of(i * 16, 16), 16)
            p = b3[dsl]
            bp = RWG * b1[dsl]
            icv = b2[dsl]
            b1[dsl] = p - bp
            b2[dsl] = bp + RIG * icv - p
            return carry

        lax.fori_loop(0, iters, nbody, 0)
        pltpu.sync_copy(b3.at[pl.ds(0, nsl)], pot_sh.at[pl.ds(nb, nsl)])
        pltpu.sync_copy(b1.at[pl.ds(0, nsl)], wp_sh.at[pl.ds(nb, nsl)])
        pltpu.sync_copy(b2.at[pl.ds(0, nsl)], ne_sh.at[pl.ds(nb, nsl)])
        pltpu.sync_copy(h_hbm.at[pl.ds(nb, nsl)], b1.at[pl.ds(0, nsl)])
        pltpu.sync_copy(b1.at[pl.ds(0, nsl)], hsh_sh.at[pl.ds(nb, nsl)])

        def zbody(i, carry):
            dsl = pl.ds(pl.multiple_of(i * 16, 16), 16)
            b1[dsl] = jnp.zeros((16,), jnp.float32)
            return carry

        lax.fori_loop(0, iters, zbody, 0)
        pltpu.sync_copy(b1.at[pl.ds(0, nsl)], slide_sh.at[pl.ds(nb, nsl)])
        pltpu.sync_copy(b1.at[pl.ds(0, nsl)], deg_sh.at[pl.ds(nb, nsl)])

    _stage(pl.multiple_of(s * NSL, 32), NSL, NSL // 16)

    @pl.when(s == 0)
    def _():
        _stage(NTB, NTAIL, NTAIL // 16)

    for i in range(CH // 16):
        ones[pl.ds(i * 16, 16)] = jnp.ones((16,), jnp.float32)

    plsc.subcore_barrier()

    n_iters = jnp.where(w < EXTRA, BASE_UNITS + 1, BASE_UNITS)
    start = BASE_UNITS * w + jnp.minimum(w, EXTRA)

    def ebody(j, carry):
        base = pl.multiple_of((start + j) * UE, UE)
        cp = [pltpu.async_copy(chan_hbm.at[pl.ds(base, UE)], sv, seml),
              pltpu.async_copy(slid_hbm.at[pl.ds(base, UE)], uv, seml)]
        for r in range(RB):
            brow = pl.ds(base + r * CH, CH)
            cp.append(pltpu.async_copy(head_hbm.at[brow], ihs[r], seml))
            cp.append(pltpu.async_copy(tail_hbm.at[brow], its[r], seml))
        for x in cp:
            x.wait()
        gs = []
        for r in range(RB):
            drow = pl.ds(r * CH, CH)
            gs += [pltpu.async_copy(pot_sh.at[ihs[r]], gph.at[drow], semg),
                   pltpu.async_copy(pot_sh.at[its[r]], gpt.at[drow], semg),
                   pltpu.async_copy(hsh_sh.at[ihs[r]], ghh.at[drow], semg),
                   pltpu.async_copy(hsh_sh.at[its[r]], ght.at[drow], semg),
                   pltpu.async_copy(wp_sh.at[ihs[r]], gwh.at[drow], semg),
                   pltpu.async_copy(wp_sh.at[its[r]], gwt.at[drow], semg),
                   pltpu.async_copy(ne_sh.at[ihs[r]], gnh.at[drow], semg),
                   pltpu.async_copy(ne_sh.at[its[r]], gnt.at[drow], semg)]
        for x in gs:
            x.wait()
        for r in range(RB):
            for i in range(CH // 16):
                dsl = pl.ds(r * CH + i * 16, 16)
                dsl_r = pl.ds(i * 16, 16)
                dsdt, aslide = _edge_math(gph[dsl], gpt[dsl],
                                          ghh[dsl], ght[dsl],
                                          gwh[dsl], gwt[dsl], gnh[dsl], gnt[dsl],
                                          sv[dsl], uv[dsl])
                dq[dsl] = dsdt
                sl[r, dsl_r] = aslide
        ss = [pltpu.async_copy(dq, dq_out.at[pl.ds(base, UE)], seml)]
        for r in range(RB):
            ss += [pltpu.async_copy(sl.at[r], slide_sh.at[ihs[r]], sems, add=True),
                   pltpu.async_copy(sl.at[r], slide_sh.at[its[r]], sems, add=True),
                   pltpu.async_copy(ones, deg_sh.at[ihs[r]], sems, add=True),
                   pltpu.async_copy(ones, deg_sh.at[its[r]], sems, add=True)]
        for x in ss:
            x.wait()
        return carry

    lax.fori_loop(0, n_iters, ebody, 0)

    plsc.subcore_barrier()

    def _wb(nb, nsl, slide_out, deg_out):
        pltpu.sync_copy(slide_sh.at[pl.ds(nb, nsl)], b1.at[pl.ds(0, nsl)])
        pltpu.sync_copy(b1.at[pl.ds(0, nsl)], slide_out.at[pl.ds(nb, nsl)])
        pltpu.sync_copy(deg_sh.at[pl.ds(nb, nsl)], b2.at[pl.ds(0, nsl)])
        pltpu.sync_copy(b2.at[pl.ds(0, nsl)], deg_out.at[pl.ds(nb, nsl)])

    nb_main = pl.multiple_of(s * NSL, 32)

    @pl.when(c == 0)
    def _():
        _wb(nb_main, NSL, sl0_out, dg0_out)

    @pl.when(c == 1)
    def _():
        _wb(nb_main, NSL, sl1_out, dg1_out)

    @pl.when((s == 0) & (c == 0))
    def _():
        _wb(NTB, NTAIL, sl0_out, dg0_out)

    @pl.when((s == 0) & (c == 1))
    def _():
        _wb(NTB, NTAIL, sl1_out, dg1_out)


def _node_body(pot_hbm, h_hbm, bed_hbm, ice_hbm, sl0_hbm, sl1_hbm, dg0_hbm, dg1_hbm,
               dh_out,
               potb, hb, bedb, iceb, sp0, sp1, dp0, dp1, dhb):
    c = lax.axis_index("c")
    s = lax.axis_index("s")
    w = s * NC + c

    def _run(nb, nsl, iters):
        pltpu.sync_copy(pot_hbm.at[pl.ds(nb, nsl)], potb.at[pl.ds(0, nsl)])
        pltpu.sync_copy(h_hbm.at[pl.ds(nb, nsl)], hb.at[pl.ds(0, nsl)])
        pltpu.sync_copy(bed_hbm.at[pl.ds(nb, nsl)], bedb.at[pl.ds(0, nsl)])
        pltpu.sync_copy(ice_hbm.at[pl.ds(nb, nsl)], iceb.at[pl.ds(0, nsl)])
        pltpu.sync_copy(sl0_hbm.at[pl.ds(nb, nsl)], sp0.at[pl.ds(0, nsl)])
        pltpu.sync_copy(sl1_hbm.at[pl.ds(nb, nsl)], sp1.at[pl.ds(0, nsl)])
        pltpu.sync_copy(dg0_hbm.at[pl.ds(nb, nsl)], dp0.at[pl.ds(0, nsl)])
        pltpu.sync_copy(dg1_hbm.at[pl.ds(nb, nsl)], dp1.at[pl.ds(0, nsl)])

        def nbody(i, carry):
            dsl = pl.ds(pl.multiple_of(i * 16, 16), 16)
            p = potb[dsl]
            h = hb[dsl]
            ne = RWG * bedb[dsl] + RIG * iceb[dsl] - p
            nec = jnp.maximum(ne, 0.0)
            scl = CLOSURE * h * (nec * nec * nec)
            dg = dp0[dsl] + dp1[dsl]
            sn = (sp0[dsl] + sp1[dsl]) / jnp.maximum(dg, 1.0)
            opening = jnp.where(h < BED_STEP,
                                sn * (BED_STEP - h) * (1.0 / CAV_SPACING), 0.0)
            dhb[dsl] = opening - scl
            return carry

        lax.fori_loop(0, iters, nbody, 0)
        pltpu.sync_copy(dhb.at[pl.ds(0, nsl)], dh_out.at[pl.ds(nb, nsl)])

    _run(pl.multiple_of(w * WSL, 16), WSL, WSL // 16)

    @pl.when(w == 0)
    def _():
        _run(WTB, WTAIL, WTAIL // 16)


_MESH = plsc.VectorSubcoreMesh(core_axis_name="c", subcore_axis_name="s")

_edge_kernel = functools.partial(
    pl.kernel,
    out_type=(jax.ShapeDtypeStruct((E,), jnp.float32),
              jax.ShapeDtypeStruct((N,), jnp.float32),
              jax.ShapeDtypeStruct((N,), jnp.float32),
              jax.ShapeDtypeStruct((N,), jnp.float32),
              jax.ShapeDtypeStruct((N,), jnp.float32)),
    mesh=_MESH,
    scratch_types=(
        pltpu.VMEM_SHARED((N,), jnp.float32),
        pltpu.VMEM_SHARED((N,), jnp.float32),
        pltpu.VMEM_SHARED((N,), jnp.float32),
        pltpu.VMEM_SHARED((N,), jnp.float32),
        pltpu.VMEM_SHARED((N,), jnp.float32),
        pltpu.VMEM_SHARED((N,), jnp.float32),
        pltpu.VMEM((NSL,), jnp.float32),
        pltpu.VMEM((NSL,), jnp.float32),
        pltpu.VMEM((NSL,), jnp.float32),
        pltpu.VMEM((CH,), jnp.int32),
        pltpu.VMEM((CH,), jnp.int32),
        pltpu.VMEM((CH,), jnp.int32),
        pltpu.VMEM((CH,), jnp.int32),
        pltpu.VMEM((CH,), jnp.int32),
        pltpu.VMEM((CH,), jnp.int32),
        pltpu.VMEM((CH,), jnp.int32),
        pltpu.VMEM((CH,), jnp.int32),
        pltpu.VMEM((UE,), jnp.float32),
        pltpu.VMEM((UE,), jnp.float32),
        pltpu.VMEM((UE,), jnp.float32),
        pltpu.VMEM((UE,), jnp.float32),
        pltpu.VMEM((UE,), jnp.float32),
        pltpu.VMEM((UE,), jnp.float32),
        pltpu.VMEM((UE,), jnp.float32),
        pltpu.VMEM((UE,), jnp.float32),
        pltpu.VMEM((UE,), jnp.float32),
        pltpu.VMEM((UE,), jnp.float32),
        pltpu.VMEM((UE,), jnp.float32),
        pltpu.VMEM((RB, CH), jnp.float32),
        pltpu.VMEM((CH,), jnp.float32),
        pltpu.SemaphoreType.DMA,
        pltpu.SemaphoreType.DMA,
        pltpu.SemaphoreType.DMA,
    ),
)(_edge_body)

_node_kernel = functools.partial(
    pl.kernel,
    out_type=jax.ShapeDtypeStruct((N,), jnp.float32),
    mesh=_MESH,
    scratch_types=tuple([pltpu.VMEM((WSL,), jnp.float32)] * 9),
)(_node_body)


def kernel(potential, sheet_thickness, channel_size, sliding_velocity,
           bedrock_elevation, ice_thickness, edge_index):
    tail = edge_index[0]
    head = edge_index[1]
    dsdt, sl0, sl1, dg0, dg1 = _edge_kernel(
        potential, sheet_thickness, bedrock_elevation, ice_thickness,
        channel_size, sliding_velocity, tail, head)
    dhdt = _node_kernel(potential, sheet_thickness, bedrock_elevation,
                        ice_thickness, sl0, sl1, dg0, dg1)
    return jnp.concatenate([dhdt, dsdt])
